# baseline (speedup 1.0000x reference)
import functools

import jax
import jax.numpy as jnp
from jax import lax
from jax.experimental import pallas as pl
from jax.experimental.pallas import tpu as pltpu
from jax.experimental.pallas import tpu_sc as plsc

_OHEM_RATIO = 3.0
_NSTAT = 6
_CPX = 4096
_HWS = (16384, 4096, 1024)
_NIMG = 8
_NCH = 28


def _f32_from_bits(i):
    return lax.bitcast_convert_type(i, jnp.float32)


def _ce_nll(la, lb, tgt):
    m = jnp.maximum(la, lb)
    lse = m + jnp.log(jnp.exp(la - m) + jnp.exp(lb - m))
    lt = jnp.where(tgt > 0.0, lb, la)
    return lse - lt


def _sc_wl_body(reg3, gt3, reg4, gt4, reg5, gt5, out,
                trb, tclb, tmb, pwb, rb0, rb1, wb0, wb1, stage,
                sem_r0, sem_r1, sem_w0, sem_w1):
    wid = lax.axis_index("s") * 2 + lax.axis_index("c")
    img = wid // 4
    ch0 = (wid % 4) * 7
    rbufs = (rb0, rb1)
    wbufs = (wb0, wb1)
    rsems = (sem_r0, sem_r1)
    wsems = (sem_w0, sem_w1)
    for lvl, (reg_h, gt_h) in enumerate(((reg3, gt3), (reg4, gt4),
                                         (reg5, gt5))):
        hw = _HWS[lvl]
        cpx = min(_CPX, hw)
        nvec = cpx // 16
        acc = jnp.zeros((16,), jnp.float32)
        for chunk in range(hw // cpx):
            base = chunk * cpx
            gb = img * 31 * hw + base
            dst = pl.ds(0, cpx)
            pltpu.sync_copy(gt_h.at[pl.ds(gb, cpx)], trb.at[dst])
            pltpu.sync_copy(gt_h.at[pl.ds(gb + hw, cpx)], tclb.at[dst])
            pltpu.sync_copy(gt_h.at[pl.ds(gb + 2 * hw, cpx)], tmb.at[dst])

            def _pw(j, carry):
                sl = pl.ds(j * 16, 16)
                t = trb[sl]
                tc = tclb[sl]
                tm = tmb[sl]
                pwb[sl] = jnp.where(t * tm > 0.0, (t + tc) * 0.2, 0.0)
                return carry

            lax.fori_loop(0, nvec, _pw, jnp.int32(0))

            def _start(c, slot):
                ch = ch0 + c
                roff = (img * _NCH + ch) * hw + base
                woff = (img * 31 + 3 + ch) * hw + base
                hr = pltpu.async_copy(
                    reg_h.at[pl.ds(roff, cpx)], rbufs[slot].at[dst],
                    rsems[slot])
                hwl = pltpu.async_copy(
                    gt_h.at[pl.ds(woff, cpx)], wbufs[slot].at[dst],
                    wsems[slot])
                return hr, hwl

            pend = _start(0, 0)
            for c in range(7):
                nxt = _start(c + 1, (c + 1) % 2) if c < 6 else None
                pend[0].wait()
                pend[1].wait()
                rb = rbufs[c % 2]
                wb = wbufs[c % 2]

                def _acc(j, a):
                    sl = pl.ds(j * 16, 16)
                    d = rb[sl] - wb[sl]
                    ad = jnp.abs(d)
                    s = jnp.where(ad < 1.0, 0.5 * d * d, ad - 0.5)
                    return a + pwb[sl] * s

                acc = lax.fori_loop(0, nvec, _acc, acc)
                pend = nxt
        stage[...] = acc
        pltpu.sync_copy(stage, out.at[lvl, wid])


def _sc_wl(reg3, gt3, reg4, gt4, reg5, gt5):
    f = pl.kernel(
        _sc_wl_body,
        out_type=jax.ShapeDtypeStruct((3, 32, 16), jnp.float32),
        mesh=plsc.VectorSubcoreMesh(core_axis_name="c", subcore_axis_name="s"),
        scratch_types=[
            pltpu.VMEM((_CPX,), jnp.float32),
            pltpu.VMEM((_CPX,), jnp.float32),
            pltpu.VMEM((_CPX,), jnp.float32),
            pltpu.VMEM((_CPX,), jnp.float32),
            pltpu.VMEM((_CPX,), jnp.float32),
            pltpu.VMEM((_CPX,), jnp.float32),
            pltpu.VMEM((_CPX,), jnp.float32),
            pltpu.VMEM((_CPX,), jnp.float32),
            pltpu.VMEM((16,), jnp.float32),
            pltpu.SemaphoreType.DMA,
            pltpu.SemaphoreType.DMA,
            pltpu.SemaphoreType.DMA,
            pltpu.SemaphoreType.DMA,
        ],
    )
    return f(reg3.reshape(-1), gt3.reshape(-1), reg4.reshape(-1),
             gt4.reshape(-1), reg5.reshape(-1), gt5.reshape(-1))


def _dense_step(cls_r, gt_r, neg_r, i):
    l0 = cls_r[0, 0]
    l1 = cls_r[0, 1]
    l2 = cls_r[0, 2]
    l3 = cls_r[0, 3]
    tr = gt_r[0, 0]
    tcl = gt_r[0, 1]
    tm = gt_r[0, 2]

    nll_tr = _ce_nll(l0, l1, tr)
    posf = jnp.where(tr * tm > 0.0, 1.0, 0.0).astype(jnp.float32)
    negf = jnp.where((1.0 - tr) * tm > 0.0, 1.0, 0.0).astype(jnp.float32)
    n_pos = jnp.sum(posf)
    n_neg = jnp.sum(negf)
    loss_pos = jnp.sum(posf * nll_tr)
    sum_neg = jnp.sum(negf * nll_tr)
    neg_r[i] = jnp.where(negf > 0.0, nll_tr, -1.0)

    nll_tcl = _ce_nll(l2, l3, tcl)
    s_tcl_pos = jnp.sum(posf * nll_tcl)
    s_tcl_neg = jnp.sum((1.0 - posf) * nll_tcl)
    return n_pos, n_neg, loss_pos, sum_neg, s_tcl_pos, s_tcl_neg


def _topk_sum(neg_r, k, n_neg, sum_neg):
    def _search(_):
        def body(_, lohi):
            lo, hi = lohi
            mid = lo + (hi - lo) // 2
            t = _f32_from_bits(mid)
            cnt = jnp.sum(jnp.where(neg_r[...] >= t, 1.0, 0.0))
            ge = cnt >= k
            return jnp.where(ge, mid, lo), jnp.where(ge, hi, mid)

        lo, _ = lax.fori_loop(
            0, 31, body, (jnp.int32(0), jnp.int32(0x7F800000)))
        v = _f32_from_bits(lo)
        arr = neg_r[...]
        gtm = jnp.where(arr > v, 1.0, 0.0)
        return jnp.sum(arr * gtm) + (k - jnp.sum(gtm)) * v

    return lax.cond(k >= n_neg, lambda _: sum_neg, _search, 0)


def _tc_body(cls3_r, gt3_r, cls4_r, gt4_r, cls5_r, gt5_r,
             out_r, neg3_r, neg4_r, neg5_r, acc_r):
    i = pl.program_id(0)
    groups = ((cls3_r, gt3_r, neg3_r),
              (cls4_r, gt4_r, neg4_r),
              (cls5_r, gt5_r, neg5_r))
    for lvl, (cls_r, gt_r, neg_r) in enumerate(groups):
        part = _dense_step(cls_r, gt_r, neg_r, i)
        for j, p in enumerate(part):
            prev = jnp.where(i > 0, acc_r[lvl, j], 0.0)
            acc_r[lvl, j] = prev + p

    @pl.when(i == pl.num_programs(0) - 1)
    def _finalize():
        ltr = jnp.float32(0.0)
        ltcl = jnp.float32(0.0)
        npos = []
        for lvl, (_, _, neg_r) in enumerate(groups):
            n_pos = acc_r[lvl, 0]
            n_neg = acc_r[lvl, 1]
            loss_pos = acc_r[lvl, 2]
            sum_neg = acc_r[lvl, 3]
            s_tcl_pos = acc_r[lvl, 4]
            s_tcl_neg = acc_r[lvl, 5]
            total = jnp.float32(
                neg_r.shape[0] * neg_r.shape[1] * neg_r.shape[2])
            cap = _OHEM_RATIO * n_pos
            nnb = jnp.minimum(n_neg, cap)
            has = n_pos > 0.0
            k = jnp.where(has, nnb, jnp.minimum(n_neg, 100.0))
            denom = jnp.where(has, n_pos + nnb, 100.0)
            s_top = _topk_sum(neg_r, k, n_neg, sum_neg)
            ltr = ltr + (loss_pos + s_top) / denom
            ltcl = ltcl + jnp.where(
                has, s_tcl_pos / n_pos + 0.5 * s_tcl_neg / (total - n_pos),
                0.0)
            npos.append(n_pos)

        lane = lax.broadcasted_iota(jnp.int32, (8, 128), 1)
        sub = lax.broadcasted_iota(jnp.int32, (8, 128), 0)
        row0 = sub == 0
        vals = (ltr, ltcl, npos[0], npos[1], npos[2])
        res = jnp.zeros((8, 128), jnp.float32)
        for j, v in enumerate(vals):
            res = res + jnp.where(row0 & (lane == j), v, 0.0)
        out_r[...] = res


def _tc_losses(cls3, gt3, cls4, gt4, cls5, gt5):
    args = []
    in_specs = []
    scratch = []
    n = _NIMG
    for c, g in ((cls3, gt3), (cls4, gt4), (cls5, gt5)):
        _, _, h, w = c.shape
        s = (h * w) // 128
        args += [c.reshape(n, 4, s, 128), g.reshape(n, 31, s, 128)]
        in_specs.append(pl.BlockSpec((1, 4, s, 128), lambda i: (i, 0, 0, 0)))
        in_specs.append(pl.BlockSpec((1, 4, s, 128), lambda i: (i, 0, 0, 0)))
        scratch.append(pltpu.VMEM((n, s, 128), jnp.float32))
    scratch.append(pltpu.SMEM((3, _NSTAT), jnp.float32))
    out = pl.pallas_call(
        _tc_body,
        grid=(n,),
        in_specs=in_specs,
        out_specs=pl.BlockSpec((8, 128), lambda i: (0, 0)),
        out_shape=jax.ShapeDtypeStruct((8, 128), jnp.float32),
        scratch_shapes=scratch,
        compiler_params=pltpu.CompilerParams(
            dimension_semantics=("arbitrary",)),
    )(*args)
    return out


def kernel(cls3, reg3, gt3, cls4, reg4, gt4, cls5, reg5, gt5):
    sc_part = _sc_wl(reg3, gt3, reg4, gt4, reg5, gt5)
    tc_out = _tc_losses(cls3, gt3, cls4, gt4, cls5, gt5)
    row = tc_out[0]
    ltr = row[0]
    ltcl = row[1]
    swl = jnp.sum(sc_part, axis=(1, 2))
    lwl = jnp.float32(0.0)
    for lvl in range(3):
        n_pos = row[2 + lvl]
        lwl = lwl + jnp.where(n_pos > 0.0, swl[lvl] / (n_pos * 28.0), 0.0)
    return jnp.stack([ltr, ltcl, lwl])


# speedup vs baseline: 1.0804x; 1.0804x over previous
import functools

import jax
import jax.numpy as jnp
from jax import lax
from jax.experimental import pallas as pl
from jax.experimental.pallas import tpu as pltpu
from jax.experimental.pallas import tpu_sc as plsc

_OHEM_RATIO = 3.0
_NSTAT = 6
_CPX = 8192
_HWS = (16384, 4096, 1024)
_NIMG = 8
_NCH = 28


def _f32_from_bits(i):
    return lax.bitcast_convert_type(i, jnp.float32)


def _ce_nll(la, lb, tgt):
    m = jnp.maximum(la, lb)
    lse = m + jnp.log(jnp.exp(la - m) + jnp.exp(lb - m))
    lt = jnp.where(tgt > 0.0, lb, la)
    return lse - lt


def _sc_wl_body(reg3, gt3, reg4, gt4, reg5, gt5, out,
                maskb, pwb, rb0, rb1, wb0, wb1, stage,
                sem_r0, sem_r1, sem_w0, sem_w1):
    wid = lax.axis_index("s") * 2 + lax.axis_index("c")
    img = wid // 4
    ch0 = (wid % 4) * 7
    rbufs = (rb0, rb1)
    wbufs = (wb0, wb1)
    rsems = (sem_r0, sem_r1)
    wsems = (sem_w0, sem_w1)
    for lvl, (reg_h, gt_h) in enumerate(((reg3, gt3), (reg4, gt4),
                                         (reg5, gt5))):
        hw = _HWS[lvl]
        cpx = min(_CPX, hw)
        pltpu.sync_copy(gt_h.at[pl.ds(img * 31 * hw, 3 * hw)],
                        maskb.at[pl.ds(0, 3 * hw)])

        def _pw(j, carry):
            sl = pl.ds(j * 16, 16)
            t = maskb[sl]
            tc = maskb[pl.ds(hw + j * 16, 16)]
            tm = maskb[pl.ds(2 * hw + j * 16, 16)]
            pwb[sl] = jnp.where(t * tm > 0.0, (t + tc) * 0.2, 0.0)
            return carry

        lax.fori_loop(0, hw // 16, _pw, jnp.int32(0))

        pairs = [(c, k) for c in range(7) for k in range(hw // cpx)]

        def _start(pair, slot):
            c, k = pair
            ch = ch0 + c
            roff = (img * _NCH + ch) * hw + k * cpx
            woff = (img * 31 + 3 + ch) * hw + k * cpx
            dst = pl.ds(0, cpx)
            hr = pltpu.async_copy(
                reg_h.at[pl.ds(roff, cpx)], rbufs[slot].at[dst], rsems[slot])
            hwl = pltpu.async_copy(
                gt_h.at[pl.ds(woff, cpx)], wbufs[slot].at[dst], wsems[slot])
            return hr, hwl

        acc = jnp.zeros((16,), jnp.float32)
        pend = _start(pairs[0], 0)
        for idx, (c, k) in enumerate(pairs):
            nxt = (_start(pairs[idx + 1], (idx + 1) % 2)
                   if idx + 1 < len(pairs) else None)
            pend[0].wait()
            pend[1].wait()
            rb = rbufs[idx % 2]
            wb = wbufs[idx % 2]
            pwoff = k * cpx

            def _acc(j, a):
                sl = pl.ds(j * 16, 16)
                d = rb[sl] - wb[sl]
                ad = jnp.abs(d)
                s = jnp.where(ad < 1.0, 0.5 * d * d, ad - 0.5)
                return a + pwb[pl.ds(pwoff + j * 16, 16)] * s

            acc = lax.fori_loop(0, cpx // 16, _acc, acc)
            pend = nxt
        stage[...] = acc
        pltpu.sync_copy(stage, out.at[lvl, wid])


def _sc_wl(reg3, gt3, reg4, gt4, reg5, gt5):
    f = pl.kernel(
        _sc_wl_body,
        out_type=jax.ShapeDtypeStruct((3, 32, 16), jnp.float32),
        mesh=plsc.VectorSubcoreMesh(core_axis_name="c", subcore_axis_name="s"),
        scratch_types=[
            pltpu.VMEM((3 * _HWS[0],), jnp.float32),
            pltpu.VMEM((_HWS[0],), jnp.float32),
            pltpu.VMEM((_CPX,), jnp.float32),
            pltpu.VMEM((_CPX,), jnp.float32),
            pltpu.VMEM((_CPX,), jnp.float32),
            pltpu.VMEM((_CPX,), jnp.float32),
            pltpu.VMEM((16,), jnp.float32),
            pltpu.SemaphoreType.DMA,
            pltpu.SemaphoreType.DMA,
            pltpu.SemaphoreType.DMA,
            pltpu.SemaphoreType.DMA,
        ],
    )
    return f(reg3.reshape(-1), gt3.reshape(-1), reg4.reshape(-1),
             gt4.reshape(-1), reg5.reshape(-1), gt5.reshape(-1))


def _dense_step(cls_r, gt_r, neg_r, i):
    l0 = cls_r[0, 0]
    l1 = cls_r[0, 1]
    l2 = cls_r[0, 2]
    l3 = cls_r[0, 3]
    tr = gt_r[0, 0]
    tcl = gt_r[0, 1]
    tm = gt_r[0, 2]

    nll_tr = _ce_nll(l0, l1, tr)
    posf = jnp.where(tr * tm > 0.0, 1.0, 0.0).astype(jnp.float32)
    negf = jnp.where((1.0 - tr) * tm > 0.0, 1.0, 0.0).astype(jnp.float32)
    n_pos = jnp.sum(posf)
    n_neg = jnp.sum(negf)
    loss_pos = jnp.sum(posf * nll_tr)
    sum_neg = jnp.sum(negf * nll_tr)
    neg_r[i] = jnp.where(negf > 0.0, nll_tr, -1.0)

    nll_tcl = _ce_nll(l2, l3, tcl)
    s_tcl_pos = jnp.sum(posf * nll_tcl)
    s_tcl_neg = jnp.sum((1.0 - posf) * nll_tcl)
    return n_pos, n_neg, loss_pos, sum_neg, s_tcl_pos, s_tcl_neg


def _topk_sum(neg_r, k, n_neg, sum_neg):
    def _search(_):
        def body(_, lohi):
            lo, hi = lohi
            mid = lo + (hi - lo) // 2
            t = _f32_from_bits(mid)
            cnt = jnp.sum(jnp.where(neg_r[...] >= t, 1.0, 0.0))
            ge = cnt >= k
            return jnp.where(ge, mid, lo), jnp.where(ge, hi, mid)

        lo, _ = lax.fori_loop(
            0, 31, body, (jnp.int32(0), jnp.int32(0x7F800000)))
        v = _f32_from_bits(lo)
        arr = neg_r[...]
        gtm = jnp.where(arr > v, 1.0, 0.0)
        return jnp.sum(arr * gtm) + (k - jnp.sum(gtm)) * v

    return lax.cond(k >= n_neg, lambda _: sum_neg, _search, 0)


def _tc_body(cls3_r, gt3_r, cls4_r, gt4_r, cls5_r, gt5_r,
             out_r, neg3_r, neg4_r, neg5_r, acc_r):
    i = pl.program_id(0)
    groups = ((cls3_r, gt3_r, neg3_r),
              (cls4_r, gt4_r, neg4_r),
              (cls5_r, gt5_r, neg5_r))
    for lvl, (cls_r, gt_r, neg_r) in enumerate(groups):
        part = _dense_step(cls_r, gt_r, neg_r, i)
        for j, p in enumerate(part):
            prev = jnp.where(i > 0, acc_r[lvl, j], 0.0)
            acc_r[lvl, j] = prev + p

    @pl.when(i == pl.num_programs(0) - 1)
    def _finalize():
        ltr = jnp.float32(0.0)
        ltcl = jnp.float32(0.0)
        npos = []
        for lvl, (_, _, neg_r) in enumerate(groups):
            n_pos = acc_r[lvl, 0]
            n_neg = acc_r[lvl, 1]
            loss_pos = acc_r[lvl, 2]
            sum_neg = acc_r[lvl, 3]
            s_tcl_pos = acc_r[lvl, 4]
            s_tcl_neg = acc_r[lvl, 5]
            total = jnp.float32(
                neg_r.shape[0] * neg_r.shape[1] * neg_r.shape[2])
            cap = _OHEM_RATIO * n_pos
            nnb = jnp.minimum(n_neg, cap)
            has = n_pos > 0.0
            k = jnp.where(has, nnb, jnp.minimum(n_neg, 100.0))
            denom = jnp.where(has, n_pos + nnb, 100.0)
            s_top = _topk_sum(neg_r, k, n_neg, sum_neg)
            ltr = ltr + (loss_pos + s_top) / denom
            ltcl = ltcl + jnp.where(
                has, s_tcl_pos / n_pos + 0.5 * s_tcl_neg / (total - n_pos),
                0.0)
            npos.append(n_pos)

        lane = lax.broadcasted_iota(jnp.int32, (8, 128), 1)
        sub = lax.broadcasted_iota(jnp.int32, (8, 128), 0)
        row0 = sub == 0
        vals = (ltr, ltcl, npos[0], npos[1], npos[2])
        res = jnp.zeros((8, 128), jnp.float32)
        for j, v in enumerate(vals):
            res = res + jnp.where(row0 & (lane == j), v, 0.0)
        out_r[...] = res


def _tc_losses(cls3, gt3, cls4, gt4, cls5, gt5):
    args = []
    in_specs = []
    scratch = []
    n = _NIMG
    for c, g in ((cls3, gt3), (cls4, gt4), (cls5, gt5)):
        _, _, h, w = c.shape
        s = (h * w) // 128
        args += [c.reshape(n, 4, s, 128), g.reshape(n, 31, s, 128)]
        in_specs.append(pl.BlockSpec((1, 4, s, 128), lambda i: (i, 0, 0, 0)))
        in_specs.append(pl.BlockSpec((1, 4, s, 128), lambda i: (i, 0, 0, 0)))
        scratch.append(pltpu.VMEM((n, s, 128), jnp.float32))
    scratch.append(pltpu.SMEM((3, _NSTAT), jnp.float32))
    out = pl.pallas_call(
        _tc_body,
        grid=(n,),
        in_specs=in_specs,
        out_specs=pl.BlockSpec((8, 128), lambda i: (0, 0)),
        out_shape=jax.ShapeDtypeStruct((8, 128), jnp.float32),
        scratch_shapes=scratch,
        compiler_params=pltpu.CompilerParams(
            dimension_semantics=("arbitrary",)),
    )(*args)
    return out


def kernel(cls3, reg3, gt3, cls4, reg4, gt4, cls5, reg5, gt5):
    sc_part = _sc_wl(reg3, gt3, reg4, gt4, reg5, gt5)
    tc_out = _tc_losses(cls3, gt3, cls4, gt4, cls5, gt5)
    row = tc_out[0]
    ltr = row[0]
    ltcl = row[1]
    swl = jnp.sum(sc_part, axis=(1, 2))
    lwl = jnp.float32(0.0)
    for lvl in range(3):
        n_pos = row[2 + lvl]
        lwl = lwl + jnp.where(n_pos > 0.0, swl[lvl] / (n_pos * 28.0), 0.0)
    return jnp.stack([ltr, ltcl, lwl])


# speedup vs baseline: 1.2912x; 1.1951x over previous
import functools

import jax
import jax.numpy as jnp
from jax import lax
from jax.experimental import pallas as pl
from jax.experimental.pallas import tpu as pltpu
from jax.experimental.pallas import tpu_sc as plsc

_OHEM_RATIO = 3.0
_NSTAT = 6
_CPX = 8192
_HWS = (16384, 4096, 1024)
_NIMG = 8
_NCH = 28


def _f32_from_bits(i):
    return lax.bitcast_convert_type(i, jnp.float32)


def _ce_nll(la, lb, tgt):
    m = jnp.maximum(la, lb)
    lse = m + jnp.log(jnp.exp(la - m) + jnp.exp(lb - m))
    lt = jnp.where(tgt > 0.0, lb, la)
    return lse - lt


def _sc_wl_body(reg3, gt3, reg4, gt4, reg5, gt5, out,
                maskb, pwb, rb0, rb1, wb0, wb1, stage,
                sem_r0, sem_r1, sem_w0, sem_w1):
    wid = lax.axis_index("s") * 2 + lax.axis_index("c")
    img = wid // 4
    ch0 = (wid % 4) * 7
    rbufs = (rb0, rb1)
    wbufs = (wb0, wb1)
    rsems = (sem_r0, sem_r1)
    wsems = (sem_w0, sem_w1)
    for lvl, (reg_h, gt_h) in enumerate(((reg3, gt3), (reg4, gt4),
                                         (reg5, gt5))):
        hw = _HWS[lvl]
        cpx = min(_CPX, hw)
        pltpu.sync_copy(gt_h.at[pl.ds(img * 31 * hw, 3 * hw)],
                        maskb.at[pl.ds(0, 3 * hw)])

        def _pw(j, carry):
            base = j * 64
            for u in range(4):
                sl = pl.ds(base + u * 16, 16)
                t = maskb[sl]
                tc = maskb[pl.ds(hw + base + u * 16, 16)]
                tm = maskb[pl.ds(2 * hw + base + u * 16, 16)]
                pwb[sl] = jnp.where(t * tm > 0.0, (t + tc) * 0.2, 0.0)
            return carry

        lax.fori_loop(0, hw // 64, _pw, jnp.int32(0))

        pairs = [(c, k) for c in range(7) for k in range(hw // cpx)]

        def _start(pair, slot):
            c, k = pair
            ch = ch0 + c
            roff = (img * _NCH + ch) * hw + k * cpx
            woff = (img * 31 + 3 + ch) * hw + k * cpx
            dst = pl.ds(0, cpx)
            hr = pltpu.async_copy(
                reg_h.at[pl.ds(roff, cpx)], rbufs[slot].at[dst], rsems[slot])
            hwl = pltpu.async_copy(
                gt_h.at[pl.ds(woff, cpx)], wbufs[slot].at[dst], wsems[slot])
            return hr, hwl

        accs = (jnp.zeros((16,), jnp.float32),) * 4
        pend = _start(pairs[0], 0)
        for idx, (c, k) in enumerate(pairs):
            nxt = (_start(pairs[idx + 1], (idx + 1) % 2)
                   if idx + 1 < len(pairs) else None)
            pend[0].wait()
            pend[1].wait()
            rb = rbufs[idx % 2]
            wb = wbufs[idx % 2]
            pwoff = k * cpx

            def _acc(j, a):
                base = j * 128
                a = list(a)
                for u in range(8):
                    sl = pl.ds(base + u * 16, 16)
                    d = rb[sl] - wb[sl]
                    ad = jnp.abs(d)
                    s = jnp.where(ad < 1.0, 0.5 * d * d, ad - 0.5)
                    a[u % 4] = a[u % 4] + pwb[
                        pl.ds(pwoff + base + u * 16, 16)] * s
                return tuple(a)

            accs = lax.fori_loop(0, cpx // 128, _acc, accs)
            pend = nxt
        stage[...] = accs[0] + accs[1] + accs[2] + accs[3]
        pltpu.sync_copy(stage, out.at[lvl, wid])


def _sc_wl(reg3, gt3, reg4, gt4, reg5, gt5):
    f = pl.kernel(
        _sc_wl_body,
        out_type=jax.ShapeDtypeStruct((3, 32, 16), jnp.float32),
        mesh=plsc.VectorSubcoreMesh(core_axis_name="c", subcore_axis_name="s"),
        scratch_types=[
            pltpu.VMEM((3 * _HWS[0],), jnp.float32),
            pltpu.VMEM((_HWS[0],), jnp.float32),
            pltpu.VMEM((_CPX,), jnp.float32),
            pltpu.VMEM((_CPX,), jnp.float32),
            pltpu.VMEM((_CPX,), jnp.float32),
            pltpu.VMEM((_CPX,), jnp.float32),
            pltpu.VMEM((16,), jnp.float32),
            pltpu.SemaphoreType.DMA,
            pltpu.SemaphoreType.DMA,
            pltpu.SemaphoreType.DMA,
            pltpu.SemaphoreType.DMA,
        ],
    )
    return f(reg3.reshape(-1), gt3.reshape(-1), reg4.reshape(-1),
             gt4.reshape(-1), reg5.reshape(-1), gt5.reshape(-1))


def _dense_step(cls_r, gt_r, neg_r, i):
    l0 = cls_r[0, 0]
    l1 = cls_r[0, 1]
    l2 = cls_r[0, 2]
    l3 = cls_r[0, 3]
    tr = gt_r[0, 0]
    tcl = gt_r[0, 1]
    tm = gt_r[0, 2]

    nll_tr = _ce_nll(l0, l1, tr)
    posf = jnp.where(tr * tm > 0.0, 1.0, 0.0).astype(jnp.float32)
    negf = jnp.where((1.0 - tr) * tm > 0.0, 1.0, 0.0).astype(jnp.float32)
    n_pos = jnp.sum(posf)
    n_neg = jnp.sum(negf)
    loss_pos = jnp.sum(posf * nll_tr)
    sum_neg = jnp.sum(negf * nll_tr)
    neg_r[i] = jnp.where(negf > 0.0, nll_tr, -1.0)

    nll_tcl = _ce_nll(l2, l3, tcl)
    s_tcl_pos = jnp.sum(posf * nll_tcl)
    s_tcl_neg = jnp.sum((1.0 - posf) * nll_tcl)
    return n_pos, n_neg, loss_pos, sum_neg, s_tcl_pos, s_tcl_neg


def _topk_sum(neg_r, k, n_neg, sum_neg):
    def _search(_):
        def body(_, lohi):
            lo, hi = lohi
            mid = lo + (hi - lo) // 2
            t = _f32_from_bits(mid)
            cnt = jnp.sum(jnp.where(neg_r[...] >= t, 1.0, 0.0))
            ge = cnt >= k
            return jnp.where(ge, mid, lo), jnp.where(ge, hi, mid)

        lo, _ = lax.fori_loop(
            0, 31, body, (jnp.int32(0), jnp.int32(0x7F800000)))
        v = _f32_from_bits(lo)
        arr = neg_r[...]
        gtm = jnp.where(arr > v, 1.0, 0.0)
        return jnp.sum(arr * gtm) + (k - jnp.sum(gtm)) * v

    return lax.cond(k >= n_neg, lambda _: sum_neg, _search, 0)


def _tc_body(cls3_r, gt3_r, cls4_r, gt4_r, cls5_r, gt5_r,
             out_r, neg3_r, neg4_r, neg5_r, acc_r):
    i = pl.program_id(0)
    groups = ((cls3_r, gt3_r, neg3_r),
              (cls4_r, gt4_r, neg4_r),
              (cls5_r, gt5_r, neg5_r))
    for lvl, (cls_r, gt_r, neg_r) in enumerate(groups):
        part = _dense_step(cls_r, gt_r, neg_r, i)
        for j, p in enumerate(part):
            prev = jnp.where(i > 0, acc_r[lvl, j], 0.0)
            acc_r[lvl, j] = prev + p

    @pl.when(i == pl.num_programs(0) - 1)
    def _finalize():
        ltr = jnp.float32(0.0)
        ltcl = jnp.float32(0.0)
        npos = []
        for lvl, (_, _, neg_r) in enumerate(groups):
            n_pos = acc_r[lvl, 0]
            n_neg = acc_r[lvl, 1]
            loss_pos = acc_r[lvl, 2]
            sum_neg = acc_r[lvl, 3]
            s_tcl_pos = acc_r[lvl, 4]
            s_tcl_neg = acc_r[lvl, 5]
            total = jnp.float32(
                neg_r.shape[0] * neg_r.shape[1] * neg_r.shape[2])
            cap = _OHEM_RATIO * n_pos
            nnb = jnp.minimum(n_neg, cap)
            has = n_pos > 0.0
            k = jnp.where(has, nnb, jnp.minimum(n_neg, 100.0))
            denom = jnp.where(has, n_pos + nnb, 100.0)
            s_top = _topk_sum(neg_r, k, n_neg, sum_neg)
            ltr = ltr + (loss_pos + s_top) / denom
            ltcl = ltcl + jnp.where(
                has, s_tcl_pos / n_pos + 0.5 * s_tcl_neg / (total - n_pos),
                0.0)
            npos.append(n_pos)

        lane = lax.broadcasted_iota(jnp.int32, (8, 128), 1)
        sub = lax.broadcasted_iota(jnp.int32, (8, 128), 0)
        row0 = sub == 0
        vals = (ltr, ltcl, npos[0], npos[1], npos[2])
        res = jnp.zeros((8, 128), jnp.float32)
        for j, v in enumerate(vals):
            res = res + jnp.where(row0 & (lane == j), v, 0.0)
        out_r[...] = res


def _tc_losses(cls3, gt3, cls4, gt4, cls5, gt5):
    args = []
    in_specs = []
    scratch = []
    n = _NIMG
    for c, g in ((cls3, gt3), (cls4, gt4), (cls5, gt5)):
        _, _, h, w = c.shape
        s = (h * w) // 128
        args += [c.reshape(n, 4, s, 128), g.reshape(n, 31, s, 128)]
        in_specs.append(pl.BlockSpec((1, 4, s, 128), lambda i: (i, 0, 0, 0)))
        in_specs.append(pl.BlockSpec((1, 4, s, 128), lambda i: (i, 0, 0, 0)))
        scratch.append(pltpu.VMEM((n, s, 128), jnp.float32))
    scratch.append(pltpu.SMEM((3, _NSTAT), jnp.float32))
    out = pl.pallas_call(
        _tc_body,
        grid=(n,),
        in_specs=in_specs,
        out_specs=pl.BlockSpec((8, 128), lambda i: (0, 0)),
        out_shape=jax.ShapeDtypeStruct((8, 128), jnp.float32),
        scratch_shapes=scratch,
        compiler_params=pltpu.CompilerParams(
            dimension_semantics=("arbitrary",)),
    )(*args)
    return out


def kernel(cls3, reg3, gt3, cls4, reg4, gt4, cls5, reg5, gt5):
    sc_part = _sc_wl(reg3, gt3, reg4, gt4, reg5, gt5)
    tc_out = _tc_losses(cls3, gt3, cls4, gt4, cls5, gt5)
    row = tc_out[0]
    ltr = row[0]
    ltcl = row[1]
    swl = jnp.sum(sc_part, axis=(1, 2))
    lwl = jnp.float32(0.0)
    for lvl in range(3):
        n_pos = row[2 + lvl]
        lwl = lwl + jnp.where(n_pos > 0.0, swl[lvl] / (n_pos * 28.0), 0.0)
    return jnp.stack([ltr, ltcl, lwl])
